# baseline (device time: 16712 ns/iter reference)
import jax
import jax.numpy as jnp
from jax import lax
from jax.experimental import pallas as pl
from jax.experimental.pallas import tpu as pltpu

N_DEV = 4
B, SQ, SKV, HQ_LOCAL, DH = 2, 128, 128, 4, 64
D_MODEL = 512


def kernel(x, Wq, K_ext, V_ext, Wo):
    def body(x_ref, wq_ref, k_hbm, v_hbm, wo_ref, out_ref,
             acc_ref, send_ref, recv_ref, k_ref, v_ref,
             send_sems, recv_sems, kv_sems):
        my_pos = lax.axis_index("i")
        partner_a = my_pos ^ 1
        partner_b = 3 - my_pos

        h0 = HQ_LOCAL * my_pos
        cp_k = pltpu.make_async_copy(
            k_hbm.at[:, :, pl.ds(h0, HQ_LOCAL), :], k_ref, kv_sems.at[0])
        cp_v = pltpu.make_async_copy(
            v_hbm.at[:, :, pl.ds(h0, HQ_LOCAL), :], v_ref, kv_sems.at[1])
        cp_k.start()
        cp_v.start()

        barrier_sem = pltpu.get_barrier_semaphore()
        for nbr in (partner_a, partner_b):
            pl.semaphore_signal(
                barrier_sem, inc=1,
                device_id=(nbr,), device_id_type=pl.DeviceIdType.MESH,
            )

        def _exchange(stage, b, partner):
            return pltpu.make_async_remote_copy(
                src_ref=send_ref.at[stage, b],
                dst_ref=recv_ref.at[stage, b],
                send_sem=send_sems.at[stage, b],
                recv_sem=recv_sems.at[stage, b],
                device_id=(partner,),
                device_id_type=pl.DeviceIdType.MESH,
            )

        rdma_a = [_exchange(0, b, partner_a) for b in range(B)]
        rdma_b = [_exchange(1, b, partner_b) for b in range(B)]

        wq = wq_ref[:, :].astype(jnp.bfloat16)
        wo = wo_ref[:, :].astype(jnp.bfloat16)

        for b in range(B):
            xb = x_ref[b, :, :].astype(jnp.bfloat16)
            qb = jnp.dot(xb, wq, preferred_element_type=jnp.float32)
            if b == 0:
                cp_k.wait()
                cp_v.wait()
            ctxs = []
            for h in range(HQ_LOCAL):
                qh = qb[:, h * DH:(h + 1) * DH].astype(jnp.bfloat16)
                kh = k_ref[b, :, h, :].astype(jnp.bfloat16)
                vh = v_ref[b, :, h, :].astype(jnp.bfloat16)
                s = jnp.dot(qh, kh.T, preferred_element_type=jnp.float32) * 0.125
                s = s - jnp.max(s, axis=-1, keepdims=True)
                w = jnp.exp(s)
                w = w / jnp.sum(w, axis=-1, keepdims=True)
                ctxs.append(jnp.dot(w.astype(jnp.bfloat16), vh,
                                    preferred_element_type=jnp.float32))
            ctx_b = jnp.concatenate(ctxs, axis=-1)
            pb = jnp.dot(ctx_b.astype(jnp.bfloat16), wo,
                         preferred_element_type=jnp.float32)
            acc_ref[b, :, :] = pb
            send_ref[0, b, :, :] = pb.astype(jnp.bfloat16)
            if b == 0:
                pl.semaphore_wait(barrier_sem, 2)
            rdma_a[b].start()

        for b in range(B):
            rdma_a[b].wait()
            acc = acc_ref[b, :, :] + recv_ref[0, b].astype(jnp.float32)
            acc_ref[b, :, :] = acc
            send_ref[1, b, :, :] = acc.astype(jnp.bfloat16)
            rdma_b[b].start()

        for b in range(B):
            rdma_b[b].wait()
            out_ref[b, :, :] = (acc_ref[b, :, :]
                                + recv_ref[1, b].astype(jnp.float32)
                                ).astype(out_ref.dtype)

    return pl.pallas_call(
        body,
        out_shape=jax.ShapeDtypeStruct((B, SQ, D_MODEL), jnp.float32),
        in_specs=[
            pl.BlockSpec(memory_space=pltpu.VMEM),
            pl.BlockSpec(memory_space=pltpu.VMEM),
            pl.BlockSpec(memory_space=pltpu.HBM),
            pl.BlockSpec(memory_space=pltpu.HBM),
            pl.BlockSpec(memory_space=pltpu.VMEM),
        ],
        out_specs=pl.BlockSpec(memory_space=pltpu.VMEM),
        scratch_shapes=[
            pltpu.VMEM((B, SQ, D_MODEL), jnp.float32),
            pltpu.VMEM((2, B, SQ, D_MODEL), jnp.bfloat16),
            pltpu.VMEM((2, B, SQ, D_MODEL), jnp.bfloat16),
            pltpu.VMEM((B, SQ, HQ_LOCAL, DH), jnp.float32),
            pltpu.VMEM((B, SQ, HQ_LOCAL, DH), jnp.float32),
            pltpu.SemaphoreType.DMA((2, B)),
            pltpu.SemaphoreType.DMA((2, B)),
            pltpu.SemaphoreType.DMA((2,)),
        ],
        compiler_params=pltpu.CompilerParams(collective_id=0),
    )(x, Wq, K_ext, V_ext, Wo)


# device time: 16384 ns/iter; 1.0200x vs baseline; 1.0200x over previous
import jax
import jax.numpy as jnp
from jax import lax
from jax.experimental import pallas as pl
from jax.experimental.pallas import tpu as pltpu

N_DEV = 4
B, SQ, SKV, HQ_LOCAL, DH = 2, 128, 128, 4, 64
D_MODEL = 512


def kernel(x, Wq, K_ext, V_ext, Wo):
    def body(x_ref, wq_ref, k_ref, v_ref, wo_ref, out_ref,
             acc_ref, send_ref, recv_ref,
             send_sems, recv_sems):
        my_pos = lax.axis_index("i")
        partner_a = my_pos ^ 1
        partner_b = 3 - my_pos

        h0 = HQ_LOCAL * my_pos

        barrier_sem = pltpu.get_barrier_semaphore()
        for nbr in (partner_a, partner_b):
            pl.semaphore_signal(
                barrier_sem, inc=1,
                device_id=(nbr,), device_id_type=pl.DeviceIdType.MESH,
            )

        def _exchange(stage, b, partner):
            return pltpu.make_async_remote_copy(
                src_ref=send_ref.at[stage, b],
                dst_ref=recv_ref.at[stage, b],
                send_sem=send_sems.at[stage, b],
                recv_sem=recv_sems.at[stage, b],
                device_id=(partner,),
                device_id_type=pl.DeviceIdType.MESH,
            )

        rdma_a = [_exchange(0, b, partner_a) for b in range(B)]
        rdma_b = [_exchange(1, b, partner_b) for b in range(B)]

        wq = wq_ref[:, :].astype(jnp.bfloat16)
        wo = wo_ref[:, :].astype(jnp.bfloat16)

        for b in range(B):
            xb = x_ref[b, :, :].astype(jnp.bfloat16)
            qb = jnp.dot(xb, wq, preferred_element_type=jnp.float32)
            ctxs = []
            for h in range(HQ_LOCAL):
                qh = qb[:, h * DH:(h + 1) * DH].astype(jnp.bfloat16)
                kh = k_ref[b, :, pl.ds(h0 + h, 1), :].reshape(
                    SKV, DH).astype(jnp.bfloat16)
                vh = v_ref[b, :, pl.ds(h0 + h, 1), :].reshape(
                    SKV, DH).astype(jnp.bfloat16)
                s = jnp.dot(qh, kh.T, preferred_element_type=jnp.float32) * 0.125
                s = s - jnp.max(s, axis=-1, keepdims=True)
                w = jnp.exp(s)
                w = w / jnp.sum(w, axis=-1, keepdims=True)
                ctxs.append(jnp.dot(w.astype(jnp.bfloat16), vh,
                                    preferred_element_type=jnp.float32))
            ctx_b = jnp.concatenate(ctxs, axis=-1)
            pb = jnp.dot(ctx_b.astype(jnp.bfloat16), wo,
                         preferred_element_type=jnp.float32)
            acc_ref[b, :, :] = pb
            send_ref[0, b, :, :] = pb.astype(jnp.bfloat16)
            if b == 0:
                pl.semaphore_wait(barrier_sem, 2)
            rdma_a[b].start()

        for b in range(B):
            rdma_a[b].wait()
            acc = acc_ref[b, :, :] + recv_ref[0, b].astype(jnp.float32)
            acc_ref[b, :, :] = acc
            send_ref[1, b, :, :] = acc.astype(jnp.bfloat16)
            rdma_b[b].start()

        for b in range(B):
            rdma_b[b].wait()
            out_ref[b, :, :] = (acc_ref[b, :, :]
                                + recv_ref[1, b].astype(jnp.float32)
                                ).astype(out_ref.dtype)

    return pl.pallas_call(
        body,
        out_shape=jax.ShapeDtypeStruct((B, SQ, D_MODEL), jnp.float32),
        in_specs=[pl.BlockSpec(memory_space=pltpu.VMEM)] * 5,
        out_specs=pl.BlockSpec(memory_space=pltpu.VMEM),
        scratch_shapes=[
            pltpu.VMEM((B, SQ, D_MODEL), jnp.float32),
            pltpu.VMEM((2, B, SQ, D_MODEL), jnp.bfloat16),
            pltpu.VMEM((2, B, SQ, D_MODEL), jnp.bfloat16),
            pltpu.SemaphoreType.DMA((2, B)),
            pltpu.SemaphoreType.DMA((2, B)),
        ],
        compiler_params=pltpu.CompilerParams(collective_id=0),
    )(x, Wq, K_ext, V_ext, Wo)
